# baseline (speedup 1.0000x reference)
import jax
import jax.numpy as jnp
from jax.experimental import pallas as pl

_B, _D, _T = 16, 64, 576
_K = 1024


def _vq_body(zt_ref, cb_ref, out_ref):
    zb = zt_ref[0]
    cb = cb_ref[...]
    m2 = jax.lax.dot_general(zb, cb, (((1,), (1,)), ((), ())))
    zz = jnp.sum(zb * zb, axis=1, keepdims=True)
    cc = jnp.sum(cb * cb, axis=1)
    dists = (zz - 2.0 * m2) + cc[None, :]
    mn = jnp.min(dists, axis=1, keepdims=True)
    kio = jax.lax.broadcasted_iota(jnp.int32, (_T, _K), 1)
    idxs = jnp.min(jnp.where(dists == mn, kio, _K), axis=1, keepdims=True)
    onehot = (kio == idxs).astype(jnp.float32)
    out_ref[0] = jax.lax.dot_general(
        cb, onehot, (((0,), (1,)), ((), ())),
        precision=jax.lax.Precision.HIGHEST)


def kernel(z, codebook):
    zt = jnp.transpose(z, (0, 2, 1))
    return pl.pallas_call(
        _vq_body,
        grid=(_B,),
        in_specs=[
            pl.BlockSpec((1, _T, _D), lambda b: (b, 0, 0)),
            pl.BlockSpec((_K, _D), lambda b: (0, 0)),
        ],
        out_specs=pl.BlockSpec((1, _D, _T), lambda b: (b, 0, 0)),
        out_shape=jax.ShapeDtypeStruct((_B, _D, _T), jnp.float32),
    )(zt, codebook)


# speedup vs baseline: 1.1219x; 1.1219x over previous
import jax
import jax.numpy as jnp
from jax.experimental import pallas as pl

_B, _D, _T = 16, 64, 576
_K = 1024


def _vq_body(z_ref, cb_ref, out_ref):
    zb = jnp.transpose(z_ref[0], (1, 0))
    cb = cb_ref[...]
    m2 = jax.lax.dot_general(zb, cb, (((1,), (1,)), ((), ())))
    zz = jnp.sum(zb * zb, axis=1, keepdims=True)
    cc = jnp.sum(cb * cb, axis=1)
    dists = (zz - 2.0 * m2) + cc[None, :]
    idxs = jnp.argmin(dists, axis=1).astype(jnp.int32)
    kio = jax.lax.broadcasted_iota(jnp.int32, (_T, _K), 1)
    onehot = (kio == idxs[:, None]).astype(jnp.float32)
    out_ref[0] = jax.lax.dot_general(
        cb, onehot, (((0,), (1,)), ((), ())),
        precision=jax.lax.Precision.HIGHEST)


def kernel(z, codebook):
    return pl.pallas_call(
        _vq_body,
        grid=(_B,),
        in_specs=[
            pl.BlockSpec((1, _D, _T), lambda b: (b, 0, 0)),
            pl.BlockSpec((_K, _D), lambda b: (0, 0)),
        ],
        out_specs=pl.BlockSpec((1, _D, _T), lambda b: (b, 0, 0)),
        out_shape=jax.ShapeDtypeStruct((_B, _D, _T), jnp.float32),
    )(z, codebook)
